# Initial kernel scaffold; baseline (speedup 1.0000x reference)
import jax
import jax.numpy as jnp
from jax.experimental import pallas as pl


def kernel(p, f):
    raise NotImplementedError("write your pallas kernel here")


# speedup vs baseline: 23.4522x; 23.4522x over previous
import functools

import jax
import jax.numpy as jnp
from jax import lax
from jax.experimental import pallas as pl
from jax.experimental.pallas import tpu as pltpu
from jax.experimental.pallas import tpu_sc as plsc

B, N, M, C, K = 16, 2048, 2048, 256, 3
TN = 512
NC, NS, L = 2, 16, 16
NW = NC * NS
P = B * N
PW = P // NW
SUB = 16
NSUB = PW // SUB


def _topk_body(xyz_ref, cxyz_ref, idx_ref, w_ref):
    b = pl.program_id(0)
    x = xyz_ref[0]
    c = cxyz_ref[0]
    xb = x.astype(jnp.bfloat16).astype(jnp.float32)
    cb = c.astype(jnp.bfloat16).astype(jnp.float32)
    x2 = (x[0:1] * x[0:1] + x[1:2] * x[1:2]) + x[2:3] * x[2:3]
    c2 = (c[:, 0:1] * c[:, 0:1] + c[:, 1:2] * c[:, 1:2]) + c[:, 2:3] * c[:, 2:3]
    dot = cb[:, 0:1] * xb[0:1]
    dot = dot + cb[:, 1:2] * xb[1:2]
    dot = dot + cb[:, 2:3] * xb[2:3]
    sq = (x2 + c2) - 2.0 * dot
    cur = jnp.sqrt(jnp.maximum(sq, 1e-12))
    iota = lax.broadcasted_iota(jnp.int32, (M, TN), 0)
    ds, js = [], []
    for t in range(K):
        m = jnp.min(cur, axis=0, keepdims=True)
        i = jnp.min(jnp.where(cur == m, iota, M), axis=0, keepdims=True)
        ds.append(m)
        js.append(i)
        if t < K - 1:
            cur = jnp.where(iota == i, jnp.float32(3.0e38), cur)
    d = jnp.concatenate(ds, axis=0)
    w = 1.0 / jnp.maximum(d, 1e-8)
    wn = w / jnp.sum(w, axis=0, keepdims=True)
    idx_ref[...] = jnp.concatenate(js, axis=0) + b * M
    w_ref[...] = wn


def _nearest_tc(xyz_t, cxyz):
    nt = N // TN
    return pl.pallas_call(
        _topk_body,
        grid=(B, nt),
        in_specs=[
            pl.BlockSpec((1, 3, TN), lambda b, n: (b, 0, n)),
            pl.BlockSpec((1, M, 3), lambda b, n: (b, 0, 0)),
        ],
        out_specs=[
            pl.BlockSpec((K, TN), lambda b, n: (0, b * (N // TN) + n)),
            pl.BlockSpec((K, TN), lambda b, n: (0, b * (N // TN) + n)),
        ],
        out_shape=[
            jax.ShapeDtypeStruct((K, P), jnp.int32),
            jax.ShapeDtypeStruct((K, P), jnp.float32),
        ],
    )(xyz_t, cxyz)


def _interp_body(table_hbm, idx_hbm, w_hbm, out_hbm, idx_v, w_v, rows_v, out_v, sem):
    wid = lax.axis_index("s") * NC + lax.axis_index("c")
    base = pl.multiple_of(wid * PW, PW)
    for j in range(K):
        pltpu.sync_copy(idx_hbm.at[pl.ds(j * P + base, PW)], idx_v.at[pl.ds(j * PW, PW)])
        pltpu.sync_copy(w_hbm.at[pl.ds(j * P + base, PW)], w_v.at[pl.ds(j * PW, PW)])

    def sub_body(s, carry):
        off = pl.multiple_of(s * SUB, SUB)
        cps = [
            pltpu.async_copy(
                table_hbm.at[idx_v[pl.ds(j * PW + off, SUB)]],
                rows_v.at[pl.ds(j * SUB, SUB)],
                sem,
            )
            for j in range(K)
        ]
        wvecs = [w_v[pl.ds(j * PW + off, SUB)] for j in range(K)]
        for cp in cps:
            cp.wait()

        for pidx in range(SUB):
            ws = [jnp.broadcast_to(lax.slice(wvecs[j], (pidx,), (pidx + 1,)), (L,))
                  for j in range(K)]
            for cc in range(C // L):
                sl = pl.ds(cc * L, L)
                out_v[pidx, sl] = (rows_v[pidx, sl] * ws[0]
                                   + rows_v[SUB + pidx, sl] * ws[1]
                                   + rows_v[2 * SUB + pidx, sl] * ws[2])

        pltpu.sync_copy(out_v, out_hbm.at[pl.ds(base + off, SUB)])
        return carry

    lax.fori_loop(0, NSUB, sub_body, 0)


@functools.cache
def _interp_sc():
    return pl.kernel(
        _interp_body,
        out_type=jax.ShapeDtypeStruct((P, C), jnp.float32),
        mesh=plsc.VectorSubcoreMesh(core_axis_name="c", subcore_axis_name="s",
                                    num_cores=NC, num_subcores=NS),
        scratch_types=[
            pltpu.VMEM((K * PW,), jnp.int32),
            pltpu.VMEM((K * PW,), jnp.float32),
            pltpu.VMEM((K * SUB, C), jnp.float32),
            pltpu.VMEM((SUB, C), jnp.float32),
            pltpu.SemaphoreType.DMA,
        ],
    )


def kernel(p, f):
    xyz_t = jnp.transpose(p[0], (0, 2, 1))
    cxyz = p[1]
    table = jnp.transpose(f[0], (0, 2, 1)).reshape(B * M, C)
    idx, w = _nearest_tc(xyz_t, cxyz)
    out = _interp_sc()(table, idx.reshape(K * P), w.reshape(K * P))
    return out.reshape(B, N, C)


# speedup vs baseline: 27.3184x; 1.1649x over previous
import functools

import jax
import jax.numpy as jnp
from jax import lax
from jax.experimental import pallas as pl
from jax.experimental.pallas import tpu as pltpu
from jax.experimental.pallas import tpu_sc as plsc

B, N, M, C, K = 16, 2048, 2048, 256, 3
TN = 512
NC, NS, L = 2, 16, 16
NW = NC * NS
P = B * N
PW = P // NW
SUB = 16
NSUB = PW // SUB


def _topk_body(xyz_ref, cxyz_ref, idx_ref, w_ref):
    b = pl.program_id(0)
    x = xyz_ref[0]
    c = cxyz_ref[0]
    xb = x.astype(jnp.bfloat16).astype(jnp.float32)
    cb = c.astype(jnp.bfloat16).astype(jnp.float32)
    x2 = (x[0:1] * x[0:1] + x[1:2] * x[1:2]) + x[2:3] * x[2:3]
    c2 = (c[:, 0:1] * c[:, 0:1] + c[:, 1:2] * c[:, 1:2]) + c[:, 2:3] * c[:, 2:3]
    dot = cb[:, 0:1] * xb[0:1]
    dot = dot + cb[:, 1:2] * xb[1:2]
    dot = dot + cb[:, 2:3] * xb[2:3]
    sq = (x2 + c2) - 2.0 * dot
    cur = jnp.sqrt(jnp.maximum(sq, 1e-12))
    iota = lax.broadcasted_iota(jnp.int32, (M, TN), 0)
    ds, js = [], []
    for t in range(K):
        m = jnp.min(cur, axis=0, keepdims=True)
        i = jnp.min(jnp.where(cur == m, iota, M), axis=0, keepdims=True)
        ds.append(m)
        js.append(i)
        if t < K - 1:
            cur = jnp.where(iota == i, jnp.float32(3.0e38), cur)
    d = jnp.concatenate(ds, axis=0)
    w = 1.0 / jnp.maximum(d, 1e-8)
    wn = w / jnp.sum(w, axis=0, keepdims=True)
    idx_ref[...] = jnp.concatenate(js, axis=0) + b * M
    w_ref[...] = wn


def _nearest_tc(xyz_t, cxyz):
    nt = N // TN
    return pl.pallas_call(
        _topk_body,
        grid=(B, nt),
        in_specs=[
            pl.BlockSpec((1, 3, TN), lambda b, n: (b, 0, n)),
            pl.BlockSpec((1, M, 3), lambda b, n: (b, 0, 0)),
        ],
        out_specs=[
            pl.BlockSpec((K, TN), lambda b, n: (0, b * (N // TN) + n)),
            pl.BlockSpec((K, TN), lambda b, n: (0, b * (N // TN) + n)),
        ],
        out_shape=[
            jax.ShapeDtypeStruct((K, P), jnp.int32),
            jax.ShapeDtypeStruct((K, P), jnp.float32),
        ],
    )(xyz_t, cxyz)


def _interp_body(table_hbm, idx_hbm, w_hbm, out_hbm,
                 idx_v, w_v, rows0, rows1, o0, o1,
                 sem_g0, sem_g1, sem_o0, sem_o1):
    wid = lax.axis_index("s") * NC + lax.axis_index("c")
    base = pl.multiple_of(wid * PW, PW)
    for j in range(K):
        pltpu.sync_copy(idx_hbm.at[pl.ds(j * P + base, PW)], idx_v.at[pl.ds(j * PW, PW)])
        pltpu.sync_copy(w_hbm.at[pl.ds(j * P + base, PW)], w_v.at[pl.ds(j * PW, PW)])

    def issue_gather(s, rows, sem):
        off = pl.multiple_of(s * SUB, SUB)
        for j in range(K):
            pltpu.async_copy(
                table_hbm.at[idx_v.at[pl.ds(j * PW + off, SUB)]],
                rows.at[pl.ds(j * SUB, SUB)], sem)

    def wait_gather(rows, sem):
        pltpu.make_async_copy(table_hbm.at[pl.ds(0, K * SUB)], rows, sem).wait()

    def wait_out(o, sem):
        pltpu.make_async_copy(o, out_hbm.at[pl.ds(base, SUB)], sem).wait()

    def issue_out(s, o, sem):
        off = pl.multiple_of(s * SUB, SUB)
        pltpu.async_copy(o, out_hbm.at[pl.ds(base + off, SUB)], sem)

    def compute(s, rows, o):
        off = pl.multiple_of(s * SUB, SUB)
        wvs = [w_v[pl.ds(j * PW + off, L)] for j in range(K)]

        dn = lax.GatherDimensionNumbers(offset_dims=(), collapsed_slice_dims=(0,),
                                        start_index_map=(0,))

        @pl.loop(0, SUB)
        def _pt(pidx):
            lane = jnp.full((L, 1), pidx, jnp.int32)
            w0, w1, w2 = (lax.gather(wv, lane, dn, slice_sizes=(1,),
                                     mode=lax.GatherScatterMode.PROMISE_IN_BOUNDS)
                          for wv in wvs)
            for cc in range(C // L):
                sl = pl.ds(cc * L, L)
                r0 = rows[pidx, sl]
                r1 = rows[SUB + pidx, sl]
                r2 = rows[2 * SUB + pidx, sl]
                o[pidx, sl] = r0 * w0 + r1 * w1 + r2 * w2

    issue_gather(0, rows0, sem_g0)

    @pl.loop(0, NSUB, step=2)
    def _ring(s0):
        wait_gather(rows0, sem_g0)
        issue_gather(s0 + 1, rows1, sem_g1)

        @pl.when(s0 >= 2)
        def _():
            wait_out(o0, sem_o0)

        compute(s0, rows0, o0)
        issue_out(s0, o0, sem_o0)

        wait_gather(rows1, sem_g1)

        @pl.when(s0 + 2 < NSUB)
        def _():
            issue_gather(s0 + 2, rows0, sem_g0)

        @pl.when(s0 >= 2)
        def _():
            wait_out(o1, sem_o1)

        compute(s0 + 1, rows1, o1)
        issue_out(s0 + 1, o1, sem_o1)

    wait_out(o0, sem_o0)
    wait_out(o1, sem_o1)


@functools.cache
def _interp_sc():
    return pl.kernel(
        _interp_body,
        out_type=jax.ShapeDtypeStruct((P, C), jnp.float32),
        mesh=plsc.VectorSubcoreMesh(core_axis_name="c", subcore_axis_name="s",
                                    num_cores=NC, num_subcores=NS),
        scratch_types=[
            pltpu.VMEM((K * PW,), jnp.int32),
            pltpu.VMEM((K * PW,), jnp.float32),
            pltpu.VMEM((K * SUB, C), jnp.float32),
            pltpu.VMEM((K * SUB, C), jnp.float32),
            pltpu.VMEM((SUB, C), jnp.float32),
            pltpu.VMEM((SUB, C), jnp.float32),
            pltpu.SemaphoreType.DMA,
            pltpu.SemaphoreType.DMA,
            pltpu.SemaphoreType.DMA,
            pltpu.SemaphoreType.DMA,
        ],
    )


def kernel(p, f):
    xyz_t = jnp.transpose(p[0], (0, 2, 1))
    cxyz = p[1]
    table = jnp.transpose(f[0], (0, 2, 1)).reshape(B * M, C)
    idx, w = _nearest_tc(xyz_t, cxyz)
    out = _interp_sc()(table, idx.reshape(K * P), w.reshape(K * P))
    return out.reshape(B, N, C)
